# baseline (speedup 1.0000x reference)
import functools

import jax
import jax.numpy as jnp
from jax import lax
from jax.experimental import pallas as pl
from jax.experimental.pallas import tpu as pltpu
from jax.experimental.pallas import tpu_sc as plsc

_EPS = 1e-5
_NC = 2
_NS = 16
_NW = _NC * _NS
_CHUNK = 120
_GPI = 4
_ROWS_IT = _CHUNK * _GPI
_NBUF = 2


def _sc_gather(table, idx_flat, t_pad):
    h = table.shape[1]
    w = t_pad // _NW
    iters = w // _ROWS_IT
    mesh = plsc.VectorSubcoreMesh(core_axis_name="c", subcore_axis_name="s")

    @functools.partial(
        pl.kernel,
        out_type=jax.ShapeDtypeStruct((t_pad, h), table.dtype),
        mesh=mesh,
        scratch_types=[
            pltpu.VMEM((_ROWS_IT,), jnp.int32),
            pltpu.VMEM((_ROWS_IT,), jnp.int32),
            pltpu.VMEM((_ROWS_IT, h), table.dtype),
            pltpu.VMEM((_ROWS_IT, h), table.dtype),
            pltpu.SemaphoreType.DMA,
            pltpu.SemaphoreType.DMA,
            pltpu.SemaphoreType.DMA,
            pltpu.SemaphoreType.DMA,
        ],
    )
    def gk(table_hbm, idx_hbm, out_hbm, idx0, idx1, rows0, rows1,
           g0, g1, w0, w1):
        wid = lax.axis_index("s") * _NC + lax.axis_index("c")
        base = wid * w
        idxs = (idx0, idx1)
        bufs = (rows0, rows1)
        gsems = (g0, g1)
        wsems = (w0, w1)

        def stage(j, b):
            pltpu.sync_copy(idx_hbm.at[pl.ds(base + j * _ROWS_IT, _ROWS_IT)],
                            idxs[b])
            for g in range(_GPI):
                pltpu.async_copy(
                    table_hbm.at[idxs[b].at[pl.ds(g * _CHUNK, _CHUNK)]],
                    bufs[b].at[pl.ds(g * _CHUNK, _CHUNK)],
                    gsems[b],
                )

        def drain_gathers(b):
            for g in range(_GPI):
                pltpu.make_async_copy(
                    table_hbm.at[pl.ds(0, _CHUNK)],
                    bufs[b].at[pl.ds(g * _CHUNK, _CHUNK)],
                    gsems[b]).wait()

        def drain_writeout(b):
            pltpu.make_async_copy(
                out_hbm.at[pl.ds(base, _ROWS_IT)], bufs[b], wsems[b]).wait()

        stage(0, 0)

        def body(jj, carry):
            for b in range(_NBUF):
                j = jj * _NBUF + b
                r0 = base + j * _ROWS_IT
                drain_gathers(b)
                pltpu.async_copy(bufs[b], out_hbm.at[pl.ds(r0, _ROWS_IT)],
                                 wsems[b])

                @pl.when(j + 1 < iters)
                def _():
                    @pl.when(j >= 1)
                    def _():
                        drain_writeout(1 - b)
                    stage(j + 1, 1 - b)
            return carry

        lax.fori_loop(0, iters // _NBUF, body, 0)
        for b in range(_NBUF):
            drain_writeout(b)

    return gk(table, idx_flat)


def _bn_affine(partials, gamma, beta, n):
    tot = jnp.sum(partials, axis=0)
    mu = tot[0:1] / n
    var = tot[1:2] / n - mu * mu
    s = gamma * lax.rsqrt(var + _EPS)
    t = beta - mu * s
    return s, t


def _part(out):
    s1 = jnp.sum(out, axis=0, keepdims=True)
    s2 = jnp.sum(out * out, axis=0, keepdims=True)
    return jnp.concatenate([s1, s2], axis=0)[None]


def _embed_body(x_ref, w_ref, b_ref, out_ref):
    out_ref[...] = (
        jnp.dot(x_ref[...], w_ref[...], preferred_element_type=jnp.float32)
        + b_ref[...]
    )


def _elu(x):
    return jnp.where(x > 0, x, jnp.exp(x) - 1.0)


def _passA_body(part_ref, gam_ref, bet_ref, a_ref, g_ref, p_ref,
                wmsg_ref, bmsg_ref, wself_ref, bself_ref, wpar_ref,
                out_ref, pout_ref, s_ref, t_ref, *, n, c_sz, k_sz):
    i = pl.program_id(0)

    @pl.when(i == 0)
    def _():
        s, t = _bn_affine(part_ref[...], gam_ref[...], bet_ref[...], n)
        s_ref[...] = s
        t_ref[...] = t

    s = s_ref[...]
    t = t_ref[...]
    h = a_ref[...] * s + t
    pn = p_ref[...].astype(jnp.float32) * s + t
    gn = [
        (g_ref[:, j, :].astype(jnp.float32) * s + t).astype(jnp.bfloat16)
        for j in range(c_sz)
    ]
    agg = jnp.zeros_like(h)
    for c in range(c_sz):
        acc = bmsg_ref[...]
        for k in range(k_sz):
            acc = acc + jnp.dot(gn[(c + k) % c_sz], wmsg_ref[k],
                                preferred_element_type=jnp.float32)
        agg = agg + _elu(acc)
    out = (
        jnp.dot(h, wself_ref[...], preferred_element_type=jnp.float32)
        + bself_ref[...]
        + jnp.dot(pn, wpar_ref[...], preferred_element_type=jnp.float32)
        + agg
        + h
    )
    out_ref[...] = out
    pout_ref[...] = _part(out)


def _passB_body(part_ref, gam_ref, bet_ref, b_ref,
                w1_ref, b1_ref, w2_ref, b2_ref,
                out_ref, pout_ref, s_ref, t_ref, *, n):
    i = pl.program_id(0)

    @pl.when(i == 0)
    def _():
        s, t = _bn_affine(part_ref[...], gam_ref[...], bet_ref[...], n)
        s_ref[...] = s
        t_ref[...] = t

    c = b_ref[...] * s_ref[...] + t_ref[...]
    u = jnp.maximum(
        jnp.dot(c, w1_ref[...], preferred_element_type=jnp.float32)
        + b1_ref[...],
        0.0,
    )
    d = (
        jnp.dot(u, w2_ref[...], preferred_element_type=jnp.float32)
        + b2_ref[...]
        + c
    )
    out_ref[...] = d
    pout_ref[...] = _part(d)


def _final_body(part_ref, gam_ref, bet_ref, b_ref, out_ref, s_ref, t_ref, *, n):
    i = pl.program_id(0)

    @pl.when(i == 0)
    def _():
        s, t = _bn_affine(part_ref[...], gam_ref[...], bet_ref[...], n)
        s_ref[...] = s
        t_ref[...] = t

    out_ref[...] = b_ref[...] * s_ref[...] + t_ref[...]


def kernel(x, edge_index, batch, circle_index, parallel_node_index,
           W_emb, b_emb, W_self, b_self, W_par, W_msg, b_msg,
           W1, b1, W2, b2, gamma1, beta1, gamma2, beta2):
    n, din = x.shape
    h_dim = W_emb.shape[1]
    n_layers = W_self.shape[0]
    c_sz = circle_index.shape[1]
    k_sz = W_msg.shape[1]
    bn = 1000
    nb = n // bn

    row = lambda i: pl.BlockSpec((bn, h_dim), lambda i_: (i_, 0))
    vec = pl.BlockSpec((1, h_dim), lambda i_: (0, 0))
    mat = pl.BlockSpec((h_dim, h_dim), lambda i_: (0, 0))
    part_in = pl.BlockSpec((nb, 2, h_dim), lambda i_: (0, 0, 0))
    part_out = pl.BlockSpec((1, 2, h_dim), lambda i_: (i_, 0, 0))
    row_f32 = jax.ShapeDtypeStruct((n, h_dim), jnp.float32)
    part_shape = jax.ShapeDtypeStruct((nb, 2, h_dim), jnp.float32)
    aff_scratch = [pltpu.VMEM((1, h_dim), jnp.float32),
                   pltpu.VMEM((1, h_dim), jnp.float32)]

    din_p = 128
    x_p = jnp.pad(x, ((0, 0), (0, din_p - din)))
    W_emb_p = jnp.pad(W_emb, ((0, din_p - din), (0, 0)))
    h0 = pl.pallas_call(
        _embed_body,
        grid=(nb,),
        in_specs=[pl.BlockSpec((bn, din_p), lambda i_: (i_, 0)),
                  pl.BlockSpec((din_p, h_dim), lambda i_: (0, 0)),
                  vec],
        out_specs=row(0),
        out_shape=row_f32,
    )(x_p, W_emb_p, b_emb[None])

    cidx = circle_index.astype(jnp.int32).reshape(-1)
    pidx = parallel_node_index.astype(jnp.int32)
    t_total = cidx.size + pidx.size
    quant = _NW * _ROWS_IT * _NBUF
    t_pad = -(-t_total // quant) * quant
    idx_flat = jnp.concatenate(
        [cidx, pidx, jnp.zeros((t_pad - t_total,), jnp.int32)])

    part = jnp.zeros((nb, 2, h_dim), jnp.float32).at[0, 1, :].set(
        n * (1.0 - _EPS))
    gam_p = jnp.ones((1, h_dim), jnp.float32)
    bet_p = jnp.zeros((1, h_dim), jnp.float32)

    a = h0
    for l in range(n_layers):
        rows = _sc_gather(a, idx_flat, t_pad)
        g_rows = rows[: n * c_sz].reshape(n, c_sz, h_dim)
        p_rows = rows[n * c_sz: n * c_sz + n]

        b_new, part1 = pl.pallas_call(
            functools.partial(_passA_body, n=n, c_sz=c_sz, k_sz=k_sz),
            grid=(nb,),
            in_specs=[part_in, vec, vec,
                      row(0),
                      pl.BlockSpec((bn, c_sz, h_dim), lambda i_: (i_, 0, 0)),
                      row(0),
                      pl.BlockSpec((k_sz, h_dim, h_dim), lambda i_: (0, 0, 0)),
                      vec, mat, vec, mat],
            out_specs=[row(0), part_out],
            out_shape=[row_f32, part_shape],
            scratch_shapes=aff_scratch,
        )(part, gam_p, bet_p, a, g_rows, p_rows,
          W_msg[l].astype(jnp.bfloat16), b_msg[l][None],
          W_self[l], b_self[l][None], W_par[l])

        a, part = pl.pallas_call(
            functools.partial(_passB_body, n=n),
            grid=(nb,),
            in_specs=[part_in, vec, vec, row(0), mat, vec, mat, vec],
            out_specs=[row(0), part_out],
            out_shape=[row_f32, part_shape],
            scratch_shapes=aff_scratch,
        )(part1, gamma1[l][None], beta1[l][None], b_new,
          W1[l], b1[l][None], W2[l], b2[l][None])

        gam_p, bet_p = gamma2[l][None], beta2[l][None]

    out = pl.pallas_call(
        functools.partial(_final_body, n=n),
        grid=(nb,),
        in_specs=[part_in, vec, vec, row(0)],
        out_specs=row(0),
        out_shape=row_f32,
        scratch_shapes=aff_scratch,
    )(part, gam_p, bet_p, a)
    return out


# speedup vs baseline: 1.0792x; 1.0792x over previous
import functools

import jax
import jax.numpy as jnp
from jax import lax
from jax.experimental import pallas as pl
from jax.experimental.pallas import tpu as pltpu
from jax.experimental.pallas import tpu_sc as plsc

_EPS = 1e-5
_NC = 2
_NS = 16
_NW = _NC * _NS
_CHUNK = 128
_GPI = 4
_ROWS_IT = _CHUNK * _GPI
_NBUF = 1


def _sc_gather(table, idx_flat, t_pad):
    h = table.shape[1]
    w = t_pad // _NW
    iters = w // _ROWS_IT
    mesh = plsc.VectorSubcoreMesh(core_axis_name="c", subcore_axis_name="s")

    @functools.partial(
        pl.kernel,
        out_type=jax.ShapeDtypeStruct((t_pad, h), table.dtype),
        mesh=mesh,
        scratch_types=[
            pltpu.VMEM((_ROWS_IT,), jnp.int32),
            pltpu.VMEM((_ROWS_IT, h), table.dtype),
            pltpu.SemaphoreType.DMA,
        ],
    )
    def gk(table_hbm, idx_hbm, out_hbm, idx_v, rows_v, sem):
        wid = lax.axis_index("s") * _NC + lax.axis_index("c")
        base = wid * w

        def body(j, carry):
            r0 = base + j * _ROWS_IT
            pltpu.sync_copy(idx_hbm.at[pl.ds(r0, _ROWS_IT)], idx_v)
            copies = [
                pltpu.async_copy(
                    table_hbm.at[idx_v.at[pl.ds(g * _CHUNK, _CHUNK)]],
                    rows_v.at[pl.ds(g * _CHUNK, _CHUNK)],
                    sem,
                )
                for g in range(_GPI)
            ]
            for c in copies:
                c.wait()
            pltpu.sync_copy(rows_v, out_hbm.at[pl.ds(r0, _ROWS_IT)])
            return carry

        lax.fori_loop(0, iters, body, 0)

    return gk(table, idx_flat)


def _bn_affine(partials, gamma, beta, n):
    tot = jnp.sum(partials, axis=0)
    mu = tot[0:1] / n
    var = tot[1:2] / n - mu * mu
    s = gamma * lax.rsqrt(var + _EPS)
    t = beta - mu * s
    return s, t


def _part(out):
    s1 = jnp.sum(out, axis=0, keepdims=True)
    s2 = jnp.sum(out * out, axis=0, keepdims=True)
    return jnp.concatenate([s1, s2], axis=0)[None]


def _embed_body(x_ref, w_ref, b_ref, out_ref):
    out_ref[...] = (
        jnp.dot(x_ref[...], w_ref[...], preferred_element_type=jnp.float32)
        + b_ref[...]
    )


def _elu(x):
    return jnp.where(x > 0, x, jnp.exp(x) - 1.0)


def _passA_body(part_ref, gam_ref, bet_ref, a_ref, g_ref, p_ref,
                wmsg_ref, bmsg_ref, wself_ref, bself_ref, wpar_ref,
                out_ref, pout_ref, s_ref, t_ref, *, n, c_sz, k_sz):
    i = pl.program_id(0)

    @pl.when(i == 0)
    def _():
        s, t = _bn_affine(part_ref[...], gam_ref[...], bet_ref[...], n)
        s_ref[...] = s
        t_ref[...] = t

    s = s_ref[...]
    t = t_ref[...]
    h = a_ref[...] * s + t
    pn = p_ref[...].astype(jnp.float32) * s + t
    gn = [
        (g_ref[:, j, :].astype(jnp.float32) * s + t).astype(jnp.bfloat16)
        for j in range(c_sz)
    ]
    agg = jnp.zeros_like(h)
    for c in range(c_sz):
        acc = bmsg_ref[...]
        for k in range(k_sz):
            acc = acc + jnp.dot(gn[(c + k) % c_sz], wmsg_ref[k],
                                preferred_element_type=jnp.float32)
        agg = agg + _elu(acc)
    out = (
        jnp.dot(h, wself_ref[...], preferred_element_type=jnp.float32)
        + bself_ref[...]
        + jnp.dot(pn, wpar_ref[...], preferred_element_type=jnp.float32)
        + agg
        + h
    )
    out_ref[...] = out
    pout_ref[...] = _part(out)


def _passB_body(part_ref, gam_ref, bet_ref, b_ref,
                w1_ref, b1_ref, w2_ref, b2_ref,
                out_ref, pout_ref, s_ref, t_ref, *, n):
    i = pl.program_id(0)

    @pl.when(i == 0)
    def _():
        s, t = _bn_affine(part_ref[...], gam_ref[...], bet_ref[...], n)
        s_ref[...] = s
        t_ref[...] = t

    c = b_ref[...] * s_ref[...] + t_ref[...]
    u = jnp.maximum(
        jnp.dot(c, w1_ref[...], preferred_element_type=jnp.float32)
        + b1_ref[...],
        0.0,
    )
    d = (
        jnp.dot(u, w2_ref[...], preferred_element_type=jnp.float32)
        + b2_ref[...]
        + c
    )
    out_ref[...] = d
    pout_ref[...] = _part(d)


def _final_body(part_ref, gam_ref, bet_ref, b_ref, out_ref, s_ref, t_ref, *, n):
    i = pl.program_id(0)

    @pl.when(i == 0)
    def _():
        s, t = _bn_affine(part_ref[...], gam_ref[...], bet_ref[...], n)
        s_ref[...] = s
        t_ref[...] = t

    out_ref[...] = b_ref[...] * s_ref[...] + t_ref[...]


def kernel(x, edge_index, batch, circle_index, parallel_node_index,
           W_emb, b_emb, W_self, b_self, W_par, W_msg, b_msg,
           W1, b1, W2, b2, gamma1, beta1, gamma2, beta2):
    n, din = x.shape
    h_dim = W_emb.shape[1]
    n_layers = W_self.shape[0]
    c_sz = circle_index.shape[1]
    k_sz = W_msg.shape[1]
    bn = 1000
    nb = n // bn

    row = lambda i: pl.BlockSpec((bn, h_dim), lambda i_: (i_, 0))
    vec = pl.BlockSpec((1, h_dim), lambda i_: (0, 0))
    mat = pl.BlockSpec((h_dim, h_dim), lambda i_: (0, 0))
    part_in = pl.BlockSpec((nb, 2, h_dim), lambda i_: (0, 0, 0))
    part_out = pl.BlockSpec((1, 2, h_dim), lambda i_: (i_, 0, 0))
    row_f32 = jax.ShapeDtypeStruct((n, h_dim), jnp.float32)
    part_shape = jax.ShapeDtypeStruct((nb, 2, h_dim), jnp.float32)
    aff_scratch = [pltpu.VMEM((1, h_dim), jnp.float32),
                   pltpu.VMEM((1, h_dim), jnp.float32)]

    din_p = 128
    x_p = jnp.pad(x, ((0, 0), (0, din_p - din)))
    W_emb_p = jnp.pad(W_emb, ((0, din_p - din), (0, 0)))
    h0 = pl.pallas_call(
        _embed_body,
        grid=(nb,),
        in_specs=[pl.BlockSpec((bn, din_p), lambda i_: (i_, 0)),
                  pl.BlockSpec((din_p, h_dim), lambda i_: (0, 0)),
                  vec],
        out_specs=row(0),
        out_shape=row_f32,
    )(x_p, W_emb_p, b_emb[None])

    cidx = circle_index.astype(jnp.int32).reshape(-1)
    pidx = parallel_node_index.astype(jnp.int32)
    t_total = cidx.size + pidx.size
    quant = _NW * _ROWS_IT * _NBUF
    t_pad = -(-t_total // quant) * quant
    idx_flat = jnp.concatenate(
        [cidx, pidx, jnp.zeros((t_pad - t_total,), jnp.int32)])

    part = jnp.zeros((nb, 2, h_dim), jnp.float32).at[0, 1, :].set(
        n * (1.0 - _EPS))
    gam_p = jnp.ones((1, h_dim), jnp.float32)
    bet_p = jnp.zeros((1, h_dim), jnp.float32)

    a = h0
    for l in range(n_layers):
        rows = _sc_gather(a, idx_flat, t_pad)
        g_rows = rows[: n * c_sz].reshape(n, c_sz, h_dim)
        p_rows = rows[n * c_sz: n * c_sz + n]

        b_new, part1 = pl.pallas_call(
            functools.partial(_passA_body, n=n, c_sz=c_sz, k_sz=k_sz),
            grid=(nb,),
            in_specs=[part_in, vec, vec,
                      row(0),
                      pl.BlockSpec((bn, c_sz, h_dim), lambda i_: (i_, 0, 0)),
                      row(0),
                      pl.BlockSpec((k_sz, h_dim, h_dim), lambda i_: (0, 0, 0)),
                      vec, mat, vec, mat],
            out_specs=[row(0), part_out],
            out_shape=[row_f32, part_shape],
            scratch_shapes=aff_scratch,
        )(part, gam_p, bet_p, a, g_rows, p_rows,
          W_msg[l].astype(jnp.bfloat16), b_msg[l][None],
          W_self[l], b_self[l][None], W_par[l])

        a, part = pl.pallas_call(
            functools.partial(_passB_body, n=n),
            grid=(nb,),
            in_specs=[part_in, vec, vec, row(0), mat, vec, mat, vec],
            out_specs=[row(0), part_out],
            out_shape=[row_f32, part_shape],
            scratch_shapes=aff_scratch,
        )(part1, gamma1[l][None], beta1[l][None], b_new,
          W1[l], b1[l][None], W2[l], b2[l][None])

        gam_p, bet_p = gamma2[l][None], beta2[l][None]

    out = pl.pallas_call(
        functools.partial(_final_body, n=n),
        grid=(nb,),
        in_specs=[part_in, vec, vec, row(0)],
        out_specs=row(0),
        out_shape=row_f32,
        scratch_shapes=aff_scratch,
    )(part, gam_p, bet_p, a)
    return out


# speedup vs baseline: 1.1720x; 1.0859x over previous
import functools

import jax
import jax.numpy as jnp
from jax import lax
from jax.experimental import pallas as pl
from jax.experimental.pallas import tpu as pltpu
from jax.experimental.pallas import tpu_sc as plsc

_EPS = 1e-5
_NC = 2
_NS = 16
_NW = _NC * _NS
_CHUNK = 128
_GPI = 4
_ROWS_IT = _CHUNK * _GPI
_NBUF = 1


def _sc_gather(table, idx_flat, t_pad):
    h = table.shape[1]
    w = t_pad // _NW
    iters = w // _ROWS_IT
    mesh = plsc.VectorSubcoreMesh(core_axis_name="c", subcore_axis_name="s")

    @functools.partial(
        pl.kernel,
        out_type=jax.ShapeDtypeStruct((t_pad, h), table.dtype),
        mesh=mesh,
        scratch_types=[
            pltpu.VMEM((_ROWS_IT,), jnp.int32),
            pltpu.VMEM((_ROWS_IT, h), table.dtype),
            pltpu.SemaphoreType.DMA,
        ],
    )
    def gk(table_hbm, idx_hbm, out_hbm, idx_v, rows_v, sem):
        wid = lax.axis_index("s") * _NC + lax.axis_index("c")
        base = wid * w

        def body(j, carry):
            r0 = base + j * _ROWS_IT
            pltpu.sync_copy(idx_hbm.at[pl.ds(r0, _ROWS_IT)], idx_v)
            copies = [
                pltpu.async_copy(
                    table_hbm.at[idx_v.at[pl.ds(g * _CHUNK, _CHUNK)]],
                    rows_v.at[pl.ds(g * _CHUNK, _CHUNK)],
                    sem,
                )
                for g in range(_GPI)
            ]
            for c in copies:
                c.wait()
            pltpu.sync_copy(rows_v, out_hbm.at[pl.ds(r0, _ROWS_IT)])
            return carry

        lax.fori_loop(0, iters, body, 0)

    return gk(table, idx_flat)


def _bn_affine(partials, gamma, beta, n):
    tot = jnp.sum(partials, axis=0)
    mu = tot[0:1] / n
    var = tot[1:2] / n - mu * mu
    s = gamma * lax.rsqrt(var + _EPS)
    t = beta - mu * s
    return s, t


def _part(out):
    s1 = jnp.sum(out, axis=0, keepdims=True)
    s2 = jnp.sum(out * out, axis=0, keepdims=True)
    return jnp.concatenate([s1, s2], axis=0)[None]


def _embed_body(x_ref, w_ref, b_ref, out_ref):
    out_ref[...] = (
        jnp.dot(x_ref[...], w_ref[...], preferred_element_type=jnp.float32)
        + b_ref[...]
    )


def _elu(x):
    return jnp.where(x > 0, x, jnp.exp(x) - 1.0)


def _passA_body(part_ref, gam_ref, bet_ref, a_ref, g_ref, p_ref,
                wmsg_ref, bmsg_ref, wself_ref, bself_ref, wpar_ref,
                out_ref, pout_ref, s_ref, t_ref, *, n, c_sz, k_sz):
    i = pl.program_id(0)

    @pl.when(i == 0)
    def _():
        s, t = _bn_affine(part_ref[...], gam_ref[...], bet_ref[...], n)
        s_ref[...] = s
        t_ref[...] = t

    s = s_ref[...]
    t = t_ref[...]
    h = a_ref[...] * s + t
    pn = p_ref[...] * s + t
    gn = [g_ref[:, j, :] * s + t for j in range(c_sz)]
    agg = jnp.zeros_like(h)
    for c in range(c_sz):
        acc = bmsg_ref[...]
        for k in range(k_sz):
            acc = acc + jnp.dot(gn[(c + k) % c_sz], wmsg_ref[k],
                                preferred_element_type=jnp.float32)
        agg = agg + _elu(acc)
    out = (
        jnp.dot(h, wself_ref[...], preferred_element_type=jnp.float32)
        + bself_ref[...]
        + jnp.dot(pn, wpar_ref[...], preferred_element_type=jnp.float32)
        + agg
        + h
    )
    out_ref[...] = out
    pout_ref[...] = _part(out)


def _passB_body(part_ref, gam_ref, bet_ref, b_ref,
                w1_ref, b1_ref, w2_ref, b2_ref,
                out_ref, pout_ref, s_ref, t_ref, *, n):
    i = pl.program_id(0)

    @pl.when(i == 0)
    def _():
        s, t = _bn_affine(part_ref[...], gam_ref[...], bet_ref[...], n)
        s_ref[...] = s
        t_ref[...] = t

    c = b_ref[...] * s_ref[...] + t_ref[...]
    u = jnp.maximum(
        jnp.dot(c, w1_ref[...], preferred_element_type=jnp.float32)
        + b1_ref[...],
        0.0,
    )
    d = (
        jnp.dot(u, w2_ref[...], preferred_element_type=jnp.float32)
        + b2_ref[...]
        + c
    )
    out_ref[...] = d
    pout_ref[...] = _part(d)


def _final_body(part_ref, gam_ref, bet_ref, b_ref, out_ref, s_ref, t_ref, *, n):
    i = pl.program_id(0)

    @pl.when(i == 0)
    def _():
        s, t = _bn_affine(part_ref[...], gam_ref[...], bet_ref[...], n)
        s_ref[...] = s
        t_ref[...] = t

    out_ref[...] = b_ref[...] * s_ref[...] + t_ref[...]


def kernel(x, edge_index, batch, circle_index, parallel_node_index,
           W_emb, b_emb, W_self, b_self, W_par, W_msg, b_msg,
           W1, b1, W2, b2, gamma1, beta1, gamma2, beta2):
    n, din = x.shape
    h_dim = W_emb.shape[1]
    n_layers = W_self.shape[0]
    c_sz = circle_index.shape[1]
    k_sz = W_msg.shape[1]
    bn = 1000
    nb = n // bn

    row = lambda i: pl.BlockSpec((bn, h_dim), lambda i_: (i_, 0))
    vec = pl.BlockSpec((1, h_dim), lambda i_: (0, 0))
    mat = pl.BlockSpec((h_dim, h_dim), lambda i_: (0, 0))
    part_in = pl.BlockSpec((nb, 2, h_dim), lambda i_: (0, 0, 0))
    part_out = pl.BlockSpec((1, 2, h_dim), lambda i_: (i_, 0, 0))
    row_f32 = jax.ShapeDtypeStruct((n, h_dim), jnp.float32)
    part_shape = jax.ShapeDtypeStruct((nb, 2, h_dim), jnp.float32)
    aff_scratch = [pltpu.VMEM((1, h_dim), jnp.float32),
                   pltpu.VMEM((1, h_dim), jnp.float32)]

    din_p = 128
    x_p = jnp.pad(x, ((0, 0), (0, din_p - din)))
    W_emb_p = jnp.pad(W_emb, ((0, din_p - din), (0, 0)))
    h0 = pl.pallas_call(
        _embed_body,
        grid=(nb,),
        in_specs=[pl.BlockSpec((bn, din_p), lambda i_: (i_, 0)),
                  pl.BlockSpec((din_p, h_dim), lambda i_: (0, 0)),
                  vec],
        out_specs=row(0),
        out_shape=row_f32,
    )(x_p, W_emb_p, b_emb[None])

    cidx = circle_index.astype(jnp.int32).reshape(-1)
    pidx = parallel_node_index.astype(jnp.int32)
    t_total = cidx.size + pidx.size
    quant = _NW * _ROWS_IT * _NBUF
    t_pad = -(-t_total // quant) * quant
    idx_flat = jnp.concatenate(
        [cidx, pidx, jnp.zeros((t_pad - t_total,), jnp.int32)])

    part = jnp.zeros((nb, 2, h_dim), jnp.float32).at[0, 1, :].set(
        n * (1.0 - _EPS))
    gam_p = jnp.ones((1, h_dim), jnp.float32)
    bet_p = jnp.zeros((1, h_dim), jnp.float32)

    a = h0
    for l in range(n_layers):
        rows = _sc_gather(a, idx_flat, t_pad)
        g_rows = rows[: n * c_sz].reshape(n, c_sz, h_dim)
        p_rows = rows[n * c_sz: n * c_sz + n]

        b_new, part1 = pl.pallas_call(
            functools.partial(_passA_body, n=n, c_sz=c_sz, k_sz=k_sz),
            grid=(nb,),
            in_specs=[part_in, vec, vec,
                      row(0),
                      pl.BlockSpec((bn, c_sz, h_dim), lambda i_: (i_, 0, 0)),
                      row(0),
                      pl.BlockSpec((k_sz, h_dim, h_dim), lambda i_: (0, 0, 0)),
                      vec, mat, vec, mat],
            out_specs=[row(0), part_out],
            out_shape=[row_f32, part_shape],
            scratch_shapes=aff_scratch,
        )(part, gam_p, bet_p, a, g_rows, p_rows,
          W_msg[l], b_msg[l][None],
          W_self[l], b_self[l][None], W_par[l])

        a, part = pl.pallas_call(
            functools.partial(_passB_body, n=n),
            grid=(nb,),
            in_specs=[part_in, vec, vec, row(0), mat, vec, mat, vec],
            out_specs=[row(0), part_out],
            out_shape=[row_f32, part_shape],
            scratch_shapes=aff_scratch,
        )(part1, gamma1[l][None], beta1[l][None], b_new,
          W1[l], b1[l][None], W2[l], b2[l][None])

        gam_p, bet_p = gamma2[l][None], beta2[l][None]

    out = pl.pallas_call(
        functools.partial(_final_body, n=n),
        grid=(nb,),
        in_specs=[part_in, vec, vec, row(0)],
        out_specs=row(0),
        out_shape=row_f32,
        scratch_shapes=aff_scratch,
    )(part, gam_p, bet_p, a)
    return out


# speedup vs baseline: 1.6881x; 1.4404x over previous
import functools

import jax
import jax.numpy as jnp
from jax import lax
from jax.experimental import pallas as pl
from jax.experimental.pallas import tpu as pltpu
from jax.experimental.pallas import tpu_sc as plsc

_EPS = 1e-5
_NC = 2
_NS = 16
_NW = _NC * _NS
_CHUNK = 128
_GPI = 4
_ROWS_IT = _CHUNK * _GPI
_NBUF = 1
_CORE0_FRAC = 0.24


def _sc_gather(table, idx_flat, t_pad):
    h = table.shape[1]
    k_tot = t_pad // (_NS * _ROWS_IT)
    k0 = max(1, int(round(k_tot * _CORE0_FRAC)))
    k1 = k_tot - k0
    mesh = plsc.VectorSubcoreMesh(core_axis_name="c", subcore_axis_name="s")

    @functools.partial(
        pl.kernel,
        out_type=jax.ShapeDtypeStruct((t_pad, h), table.dtype),
        mesh=mesh,
        scratch_types=[
            pltpu.VMEM((_ROWS_IT,), jnp.int32),
            pltpu.VMEM((_ROWS_IT, h), table.dtype),
            pltpu.SemaphoreType.DMA,
        ],
    )
    def gk(table_hbm, idx_hbm, out_hbm, idx_v, rows_v, sem):
        c = lax.axis_index("c")
        s = lax.axis_index("s")
        base = jnp.where(c == 0, s * k0, _NS * k0 + s * k1) * _ROWS_IT
        iters = jnp.where(c == 0, k0, k1)

        def body(j, carry):
            r0 = base + j * _ROWS_IT
            pltpu.sync_copy(idx_hbm.at[pl.ds(r0, _ROWS_IT)], idx_v)
            copies = [
                pltpu.async_copy(
                    table_hbm.at[idx_v.at[pl.ds(g * _CHUNK, _CHUNK)]],
                    rows_v.at[pl.ds(g * _CHUNK, _CHUNK)],
                    sem,
                )
                for g in range(_GPI)
            ]
            for c in copies:
                c.wait()
            pltpu.sync_copy(rows_v, out_hbm.at[pl.ds(r0, _ROWS_IT)])
            return carry

        lax.fori_loop(0, iters, body, 0)

    return gk(table, idx_flat)


def _bn_affine(partials, gamma, beta, n):
    tot = jnp.sum(partials, axis=0)
    mu = tot[0:1] / n
    var = tot[1:2] / n - mu * mu
    s = gamma * lax.rsqrt(var + _EPS)
    t = beta - mu * s
    return s, t


def _part(out):
    s1 = jnp.sum(out, axis=0, keepdims=True)
    s2 = jnp.sum(out * out, axis=0, keepdims=True)
    return jnp.concatenate([s1, s2], axis=0)[None]


def _embed_body(x_ref, w_ref, b_ref, out_ref):
    out_ref[...] = (
        jnp.dot(x_ref[...], w_ref[...], preferred_element_type=jnp.float32)
        + b_ref[...]
    )


def _elu(x):
    return jnp.where(x > 0, x, jnp.exp(x) - 1.0)


def _passA_body(part_ref, gam_ref, bet_ref, a_ref, g_ref, p_ref,
                wmsg_ref, bmsg_ref, wself_ref, bself_ref, wpar_ref,
                out_ref, pout_ref, s_ref, t_ref, *, n, c_sz, k_sz):
    i = pl.program_id(0)

    @pl.when(i == 0)
    def _():
        s, t = _bn_affine(part_ref[...], gam_ref[...], bet_ref[...], n)
        s_ref[...] = s
        t_ref[...] = t

    s = s_ref[...]
    t = t_ref[...]
    h = a_ref[...] * s + t
    pn = p_ref[...] * s + t
    gn = [g_ref[:, j, :] * s + t for j in range(c_sz)]
    agg = jnp.zeros_like(h)
    for c in range(c_sz):
        acc = bmsg_ref[...]
        for k in range(k_sz):
            acc = acc + jnp.dot(gn[(c + k) % c_sz], wmsg_ref[k],
                                preferred_element_type=jnp.float32)
        agg = agg + _elu(acc)
    out = (
        jnp.dot(h, wself_ref[...], preferred_element_type=jnp.float32)
        + bself_ref[...]
        + jnp.dot(pn, wpar_ref[...], preferred_element_type=jnp.float32)
        + agg
        + h
    )
    out_ref[...] = out
    pout_ref[...] = _part(out)


def _passB_body(part_ref, gam_ref, bet_ref, b_ref,
                w1_ref, b1_ref, w2_ref, b2_ref,
                out_ref, pout_ref, s_ref, t_ref, *, n):
    i = pl.program_id(0)

    @pl.when(i == 0)
    def _():
        s, t = _bn_affine(part_ref[...], gam_ref[...], bet_ref[...], n)
        s_ref[...] = s
        t_ref[...] = t

    c = b_ref[...] * s_ref[...] + t_ref[...]
    u = jnp.maximum(
        jnp.dot(c, w1_ref[...], preferred_element_type=jnp.float32)
        + b1_ref[...],
        0.0,
    )
    d = (
        jnp.dot(u, w2_ref[...], preferred_element_type=jnp.float32)
        + b2_ref[...]
        + c
    )
    out_ref[...] = d
    pout_ref[...] = _part(d)


def _final_body(part_ref, gam_ref, bet_ref, b_ref, out_ref, s_ref, t_ref, *, n):
    i = pl.program_id(0)

    @pl.when(i == 0)
    def _():
        s, t = _bn_affine(part_ref[...], gam_ref[...], bet_ref[...], n)
        s_ref[...] = s
        t_ref[...] = t

    out_ref[...] = b_ref[...] * s_ref[...] + t_ref[...]


def kernel(x, edge_index, batch, circle_index, parallel_node_index,
           W_emb, b_emb, W_self, b_self, W_par, W_msg, b_msg,
           W1, b1, W2, b2, gamma1, beta1, gamma2, beta2):
    n, din = x.shape
    h_dim = W_emb.shape[1]
    n_layers = W_self.shape[0]
    c_sz = circle_index.shape[1]
    k_sz = W_msg.shape[1]
    bn = 1000
    nb = n // bn

    row = lambda i: pl.BlockSpec((bn, h_dim), lambda i_: (i_, 0))
    vec = pl.BlockSpec((1, h_dim), lambda i_: (0, 0))
    mat = pl.BlockSpec((h_dim, h_dim), lambda i_: (0, 0))
    part_in = pl.BlockSpec((nb, 2, h_dim), lambda i_: (0, 0, 0))
    part_out = pl.BlockSpec((1, 2, h_dim), lambda i_: (i_, 0, 0))
    row_f32 = jax.ShapeDtypeStruct((n, h_dim), jnp.float32)
    part_shape = jax.ShapeDtypeStruct((nb, 2, h_dim), jnp.float32)
    aff_scratch = [pltpu.VMEM((1, h_dim), jnp.float32),
                   pltpu.VMEM((1, h_dim), jnp.float32)]

    din_p = 128
    x_p = jnp.pad(x, ((0, 0), (0, din_p - din)))
    W_emb_p = jnp.pad(W_emb, ((0, din_p - din), (0, 0)))
    h0 = pl.pallas_call(
        _embed_body,
        grid=(nb,),
        in_specs=[pl.BlockSpec((bn, din_p), lambda i_: (i_, 0)),
                  pl.BlockSpec((din_p, h_dim), lambda i_: (0, 0)),
                  vec],
        out_specs=row(0),
        out_shape=row_f32,
    )(x_p, W_emb_p, b_emb[None])

    cidx = circle_index.astype(jnp.int32).reshape(-1)
    pidx = parallel_node_index.astype(jnp.int32)
    t_total = cidx.size + pidx.size
    quant = _NS * _ROWS_IT
    t_pad = -(-t_total // quant) * quant
    idx_flat = jnp.concatenate(
        [cidx, pidx, jnp.zeros((t_pad - t_total,), jnp.int32)])

    part = jnp.zeros((nb, 2, h_dim), jnp.float32).at[0, 1, :].set(
        n * (1.0 - _EPS))
    gam_p = jnp.ones((1, h_dim), jnp.float32)
    bet_p = jnp.zeros((1, h_dim), jnp.float32)

    a = h0
    for l in range(n_layers):
        rows = _sc_gather(a, idx_flat, t_pad)
        g_rows = rows[: n * c_sz].reshape(n, c_sz, h_dim)
        p_rows = rows[n * c_sz: n * c_sz + n]

        b_new, part1 = pl.pallas_call(
            functools.partial(_passA_body, n=n, c_sz=c_sz, k_sz=k_sz),
            grid=(nb,),
            in_specs=[part_in, vec, vec,
                      row(0),
                      pl.BlockSpec((bn, c_sz, h_dim), lambda i_: (i_, 0, 0)),
                      row(0),
                      pl.BlockSpec((k_sz, h_dim, h_dim), lambda i_: (0, 0, 0)),
                      vec, mat, vec, mat],
            out_specs=[row(0), part_out],
            out_shape=[row_f32, part_shape],
            scratch_shapes=aff_scratch,
        )(part, gam_p, bet_p, a, g_rows, p_rows,
          W_msg[l], b_msg[l][None],
          W_self[l], b_self[l][None], W_par[l])

        a, part = pl.pallas_call(
            functools.partial(_passB_body, n=n),
            grid=(nb,),
            in_specs=[part_in, vec, vec, row(0), mat, vec, mat, vec],
            out_specs=[row(0), part_out],
            out_shape=[row_f32, part_shape],
            scratch_shapes=aff_scratch,
        )(part1, gamma1[l][None], beta1[l][None], b_new,
          W1[l], b1[l][None], W2[l], b2[l][None])

        gam_p, bet_p = gamma2[l][None], beta2[l][None]

    out = pl.pallas_call(
        functools.partial(_final_body, n=n),
        grid=(nb,),
        in_specs=[part_in, vec, vec, row(0)],
        out_specs=row(0),
        out_shape=row_f32,
        scratch_shapes=aff_scratch,
    )(part, gam_p, bet_p, a)
    return out
